# Initial kernel scaffold; baseline (speedup 1.0000x reference)
import jax
import jax.numpy as jnp
from jax.experimental import pallas as pl


def kernel(x, edge_index, W1, b1, W2, b2):
    raise NotImplementedError("write your pallas kernel here")


# speedup vs baseline: 3.4632x; 3.4632x over previous
import functools

import jax
import jax.numpy as jnp
from jax import lax
from jax.experimental import pallas as pl
from jax.experimental.pallas import tpu as pltpu
from jax.experimental.pallas import tpu_sc as plsc

N = 50000
D = 64
E = 800000

NC = 2
NS = 16
L = 16

PR = 25000
P = 25088
GAP = P - PR
NP = NC * P

K = 128
EP_TILE = 50048
NCHUNK = EP_TILE // K
E_PAD = EP_TILE * NS
BIG = jnp.int32(1 << 28)

BR = 128
ROWS_PER_TILE = P // NS


def _mm_body(x_ref, w_ref, o_ref, *, act):
    x = x_ref[...]
    if act:
        x = x * jax.nn.sigmoid(x)
    o_ref[...] = jnp.dot(x, w_ref[...], preferred_element_type=jnp.float32)


def _matmul(x, w, act):
    m = x.shape[0]
    bm = 512
    return pl.pallas_call(
        functools.partial(_mm_body, act=act),
        grid=(m // bm,),
        in_specs=[
            pl.BlockSpec((bm, D), lambda i: (i, 0)),
            pl.BlockSpec((D, D), lambda i: (0, 0)),
        ],
        out_specs=pl.BlockSpec((bm, D), lambda i: (i, 0)),
        out_shape=jax.ShapeDtypeStruct((m, D), jnp.float32),
    )(x, w)


def _sc_body(src_hbm, dst_hbm, m_hbm, bias_hbm, out_hbm,
             idx_src, idx_dst, rows_v, blk_v, bias_v, sem_g, sem_s, acc):
    c = lax.axis_index("c")
    s = lax.axis_index("s")
    c_p = c * P

    pltpu.sync_copy(bias_hbm, bias_v)
    bvecs = [bias_v[pl.ds(l * L, L)] for l in range(D // L)]

    def fill_row(r, carry):
        for l in range(D // L):
            blk_v[r, pl.ds(l * L, L)] = bvecs[l]
        return carry

    lax.fori_loop(0, BR, fill_row, 0)

    base = s * ROWS_PER_TILE
    for j in range(ROWS_PER_TILE // BR):
        pltpu.sync_copy(blk_v, acc.at[pl.ds(base + j * BR, BR)])
    rem = ROWS_PER_TILE % BR
    if rem:
        pltpu.sync_copy(blk_v.at[pl.ds(0, rem)],
                        acc.at[pl.ds(base + (ROWS_PER_TILE // BR) * BR, rem)])
    plsc.subcore_barrier()

    ebase = s * EP_TILE
    lane = lax.iota(jnp.int32, L)

    def chunk(i, carry):
        off = ebase + i * K
        pltpu.sync_copy(src_hbm.at[pl.ds(off, K)], idx_src.at[0])
        pltpu.sync_copy(dst_hbm.at[pl.ds(off, K)], idx_dst.at[0])
        for v in range(K // L):
            sl = pl.ds(v * L, L)
            sv = idx_src[0, sl]
            sv = sv + jnp.where(sv >= PR, GAP, 0)
            idx_src[0, sl] = sv
            dv = idx_dst[0, sl]
            dv = dv + jnp.where(dv >= PR, GAP, 0) - c_p
            valid = (dv >= 0) & (dv < PR)
            dummy = (PR + 24 + v * 8) + (lane & 7)
            dv = jnp.where(valid, dv, dummy)
            idx_dst[0, sl] = dv
        pltpu.async_copy(m_hbm.at[idx_src.at[0]], rows_v, sem_g).wait()
        pltpu.async_copy(rows_v, acc.at[idx_dst.at[0]], sem_s, add=True).wait()
        return carry

    lax.fori_loop(0, NCHUNK, chunk, 0)
    plsc.subcore_barrier()

    pltpu.sync_copy(acc.at[pl.ds(base, ROWS_PER_TILE)],
                    out_hbm.at[pl.ds(c_p + base, ROWS_PER_TILE)])


_sc_scatter = functools.partial(
    pl.kernel,
    out_type=jax.ShapeDtypeStruct((NP, D), jnp.float32),
    mesh=plsc.VectorSubcoreMesh(
        core_axis_name="c", subcore_axis_name="s", num_cores=NC,
        num_subcores=NS),
    compiler_params=pltpu.CompilerParams(use_tc_tiling_on_sc=False),
    scratch_types=[
        pltpu.VMEM((1, K), jnp.int32),
        pltpu.VMEM((1, K), jnp.int32),
        pltpu.VMEM((K, D), jnp.float32),
        pltpu.VMEM((BR, D), jnp.float32),
        pltpu.VMEM((D,), jnp.float32),
        pltpu.SemaphoreType.DMA,
        pltpu.SemaphoreType.DMA,
        pltpu.VMEM_SHARED((P, D), jnp.float32),
    ],
)(_sc_body)


def kernel(x, edge_index, W1, b1, W2, b2):
    src = edge_index[0].astype(jnp.int32)
    dst = edge_index[1].astype(jnp.int32)
    pad = E_PAD - E
    src_e = jnp.concatenate([src, jnp.zeros((pad,), jnp.int32)])
    dst_e = jnp.concatenate([dst, jnp.full((pad,), BIG, jnp.int32)])
    zgap = jnp.zeros((GAP, D), jnp.float32)
    x_p = jnp.concatenate([x[:PR], zgap, x[PR:], zgap], axis=0)

    m1 = _matmul(x_p, W1, act=False)
    agg1 = _sc_scatter(src_e, dst_e, m1, b1)
    m2 = _matmul(agg1, W2, act=True)
    agg2 = _sc_scatter(src_e, dst_e, m2, b2)
    return jnp.concatenate([agg2[:PR], agg2[P:P + PR]], axis=0)


# speedup vs baseline: 7.8116x; 2.2556x over previous
import functools

import jax
import jax.numpy as jnp
from jax import lax
from jax.experimental import pallas as pl
from jax.experimental.pallas import tpu as pltpu
from jax.experimental.pallas import tpu_sc as plsc

N = 50000
D = 64
E = 800000

NC = 2
NS = 16
L = 16

PR = 25000
P = 25088
GAP = P - PR
NP = NC * P

K = 128
NCHUNK = 392
EP_TILE = NCHUNK * K
E_PAD = EP_TILE * NS
BIG = jnp.int32(1 << 28)

NB = 3
GD = 1

BR = 64
ROWS_PER_TILE = P // NS


def _mm_body(x_ref, w_ref, o_ref, *, act):
    x = x_ref[...]
    if act:
        x = x * jax.nn.sigmoid(x)
    o_ref[...] = jnp.dot(x, w_ref[...], preferred_element_type=jnp.float32)


def _matmul(x, w, act):
    m = x.shape[0]
    bm = 512
    return pl.pallas_call(
        functools.partial(_mm_body, act=act),
        grid=(m // bm,),
        in_specs=[
            pl.BlockSpec((bm, D), lambda i: (i, 0)),
            pl.BlockSpec((D, D), lambda i: (0, 0)),
        ],
        out_specs=pl.BlockSpec((bm, D), lambda i: (i, 0)),
        out_shape=jax.ShapeDtypeStruct((m, D), jnp.float32),
    )(x, w)


def _sc_body(src_hbm, dst_hbm, m_hbm, bias_hbm, out_hbm,
             idx_src, idx_dst, rows_v, blk_v, bias_v,
             sem_i, sem_g, sem_s, acc):
    c = lax.axis_index("c")
    s = lax.axis_index("s")
    c_p = c * P

    pltpu.sync_copy(bias_hbm, bias_v)
    bvecs = [bias_v[pl.ds(l * L, L)] for l in range(D // L)]

    def fill_row(r, carry):
        for l in range(D // L):
            blk_v[r, pl.ds(l * L, L)] = bvecs[l]
        return carry

    lax.fori_loop(0, BR, fill_row, 0)

    base = s * ROWS_PER_TILE
    for j in range(ROWS_PER_TILE // BR):
        pltpu.sync_copy(blk_v, acc.at[pl.ds(base + j * BR, BR)])
    rem = ROWS_PER_TILE % BR
    if rem:
        pltpu.sync_copy(blk_v.at[pl.ds(0, rem)],
                        acc.at[pl.ds(base + (ROWS_PER_TILE // BR) * BR, rem)])
    plsc.subcore_barrier()

    ebase = s * EP_TILE
    lane = lax.iota(jnp.int32, L)

    def idx_copies(chunk, b):
        off = ebase + chunk * K
        return (pltpu.make_async_copy(src_hbm.at[pl.ds(off, K)],
                                      idx_src.at[b], sem_i.at[b]),
                pltpu.make_async_copy(dst_hbm.at[pl.ds(off, K)],
                                      idx_dst.at[b], sem_i.at[b]))

    def gather_copy(b):
        return pltpu.make_async_copy(m_hbm.at[idx_src.at[b]],
                                     rows_v.at[b], sem_g.at[b])

    def scatter_copy(b):
        return pltpu.async_copy(rows_v.at[b], acc.at[idx_dst.at[b]],
                                sem_s.at[b], add=True)

    def scatter_wait(b):
        pltpu.make_async_copy(rows_v.at[b], acc.at[idx_dst.at[b]],
                              sem_s.at[b]).wait()

    def step(i, carry):
        b = lax.rem(i, NB)

        @pl.when((i >= NB) & (i < NCHUNK))
        def _s1():
            scatter_wait(b)

        @pl.when(i < NCHUNK)
        def _s2():
            ca, cb = idx_copies(i, b)
            ca.start()
            cb.start()

        g = i - 1
        bg = lax.rem(g + NB, NB)

        @pl.when((i >= 1) & (i < NCHUNK + 1))
        def _s3():
            ca, cb = idx_copies(g, bg)
            ca.wait()
            cb.wait()
            for v in range(K // L):
                sl = pl.ds(v * L, L)
                sv = idx_src[bg, sl]
                sv = sv + jnp.where(sv >= PR, GAP, 0)
                idx_src[bg, sl] = sv
                dv = idx_dst[bg, sl]
                dv = dv + jnp.where(dv >= PR, GAP, 0) - c_p
                valid = (dv >= 0) & (dv < PR)
                dummy = (PR + 24 + v * 8) + (lane & 7)
                dv = jnp.where(valid, dv, dummy)
                idx_dst[bg, sl] = dv
            gather_copy(bg).start()

        k = i - 1 - GD
        bk = lax.rem(k + NB, NB)

        @pl.when((i >= 1 + GD) & (i < NCHUNK + 1 + GD))
        def _s4():
            gather_copy(bk).wait()
            scatter_copy(bk)

        return carry

    lax.fori_loop(0, NCHUNK + 1 + GD, step, 0)

    for b in range(NB):
        scatter_wait(b)
    plsc.subcore_barrier()

    pltpu.sync_copy(acc.at[pl.ds(base, ROWS_PER_TILE)],
                    out_hbm.at[pl.ds(c_p + base, ROWS_PER_TILE)])


_sc_scatter = functools.partial(
    pl.kernel,
    out_type=jax.ShapeDtypeStruct((NP, D), jnp.float32),
    mesh=plsc.VectorSubcoreMesh(
        core_axis_name="c", subcore_axis_name="s", num_cores=NC,
        num_subcores=NS),
    compiler_params=pltpu.CompilerParams(use_tc_tiling_on_sc=False),
    scratch_types=[
        pltpu.VMEM((NB, K), jnp.int32),
        pltpu.VMEM((NB, K), jnp.int32),
        pltpu.VMEM((NB, K, D), jnp.float32),
        pltpu.VMEM((BR, D), jnp.float32),
        pltpu.VMEM((D,), jnp.float32),
        pltpu.SemaphoreType.DMA((NB,)),
        pltpu.SemaphoreType.DMA((NB,)),
        pltpu.SemaphoreType.DMA((NB,)),
        pltpu.VMEM_SHARED((P, D), jnp.float32),
    ],
)(_sc_body)


def kernel(x, edge_index, W1, b1, W2, b2):
    src = edge_index[0].astype(jnp.int32)
    dst = edge_index[1].astype(jnp.int32)
    pad = E_PAD - E
    pad_src = PR + jnp.arange(pad, dtype=jnp.int32) % GAP
    src_e = jnp.concatenate([src, pad_src])
    dst_e = jnp.concatenate([dst, jnp.full((pad,), BIG, jnp.int32)])
    zgap = jnp.zeros((GAP, D), jnp.float32)
    x_p = jnp.concatenate([x[:PR], zgap, x[PR:], zgap], axis=0)

    m1 = _matmul(x_p, W1, act=False)
    agg1 = _sc_scatter(src_e, dst_e, m1, b1)
    m2 = _matmul(agg1, W2, act=True)
    agg2 = _sc_scatter(src_e, dst_e, m2, b2)
    return jnp.concatenate([agg2[:PR], agg2[P:P + PR]], axis=0)


# speedup vs baseline: 8.3858x; 1.0735x over previous
import functools

import jax
import jax.numpy as jnp
from jax import lax
from jax.experimental import pallas as pl
from jax.experimental.pallas import tpu as pltpu
from jax.experimental.pallas import tpu_sc as plsc

N = 50000
D = 64
E = 800000

NC = 2
NS = 16
L = 16

PR = 25000
P = 25088
GAP = P - PR
NP = NC * P

K = 128
EP0 = 50048
NCH0 = EP0 // K
E_PAD = EP0 * NS
BIG = jnp.int32(1 << 28)

NB = 3
GD = 1

BR = 64
ROWS_PER_TILE = P // NS
OUT_LAST = PR - 15 * ROWS_PER_TILE


def _mm_body(x_ref, w_ref, o_ref, *, act):
    x = x_ref[...]
    if act:
        x = x * jax.nn.sigmoid(x)
    o_ref[...] = jnp.dot(x, w_ref[...], preferred_element_type=jnp.float32)


def _matmul(x, w, act, bm):
    m = x.shape[0]
    return pl.pallas_call(
        functools.partial(_mm_body, act=act),
        grid=(m // bm,),
        in_specs=[
            pl.BlockSpec((bm, D), lambda i: (i, 0)),
            pl.BlockSpec((D, D), lambda i: (0, 0)),
        ],
        out_specs=pl.BlockSpec((bm, D), lambda i: (i, 0)),
        out_shape=jax.ShapeDtypeStruct((m, D), jnp.float32),
    )(x, w)


def _sc_body(src_hbm, dst_hbm, m_hbm, bias_hbm, out_hbm,
             idx_src, idx_dst, rows_v, blk_v, bias_v,
             sem_i, sem_g, sem_s, acc, *, remap_src, out_padded):
    c = lax.axis_index("c")
    s = lax.axis_index("s")
    c_p = c * P

    pltpu.sync_copy(bias_hbm, bias_v)
    bvecs = [bias_v[pl.ds(l * L, L)] for l in range(D // L)]

    def fill_row(r, carry):
        for l in range(D // L):
            blk_v[r, pl.ds(l * L, L)] = bvecs[l]
        return carry

    lax.fori_loop(0, BR, fill_row, 0)

    base = s * ROWS_PER_TILE
    for j in range(ROWS_PER_TILE // BR):
        pltpu.sync_copy(blk_v, acc.at[pl.ds(base + j * BR, BR)])
    rem = ROWS_PER_TILE % BR
    if rem:
        pltpu.sync_copy(blk_v.at[pl.ds(0, rem)],
                        acc.at[pl.ds(base + (ROWS_PER_TILE // BR) * BR, rem)])
    plsc.subcore_barrier()

    ebase = s * EP0
    nch = NCH0
    lane = lax.iota(jnp.int32, L)

    def idx_copies(chunk, b):
        off = ebase + chunk * K
        return (pltpu.make_async_copy(src_hbm.at[pl.ds(off, K)],
                                      idx_src.at[b], sem_i.at[b]),
                pltpu.make_async_copy(dst_hbm.at[pl.ds(off, K)],
                                      idx_dst.at[b], sem_i.at[b]))

    def gather_copy(b):
        return pltpu.make_async_copy(m_hbm.at[idx_src.at[b]],
                                     rows_v.at[b], sem_g.at[b])

    def scatter_copy(b):
        return pltpu.async_copy(rows_v.at[b], acc.at[idx_dst.at[b]],
                                sem_s.at[b], add=True)

    def scatter_wait(b):
        pltpu.make_async_copy(rows_v.at[b], acc.at[idx_dst.at[b]],
                              sem_s.at[b]).wait()

    def step(i, carry):
        b = lax.rem(i, NB)

        @pl.when((i >= NB) & (i < nch))
        def _s1():
            scatter_wait(b)

        @pl.when(i < nch)
        def _s2():
            ca, cb = idx_copies(i, b)
            ca.start()
            cb.start()

        g = i - 1
        bg = lax.rem(g + NB, NB)

        @pl.when((i >= 1) & (i < nch + 1))
        def _s3():
            ca, cb = idx_copies(g, bg)
            ca.wait()
            cb.wait()
            for v in range(K // L):
                sl = pl.ds(v * L, L)
                if remap_src:
                    sv = idx_src[bg, sl]
                    sv = sv + jnp.where(sv >= PR, GAP, 0)
                    idx_src[bg, sl] = sv
                dv = idx_dst[bg, sl]
                dv = dv + jnp.where(dv >= PR, GAP, 0) - c_p
                valid = (dv >= 0) & (dv < PR)
                dummy = (PR + 24 + v * 8) + (lane & 7)
                dv = jnp.where(valid, dv, dummy)
                idx_dst[bg, sl] = dv
            gather_copy(bg).start()

        k = i - 1 - GD
        bk = lax.rem(k + NB, NB)

        @pl.when((i >= 1 + GD) & (i < nch + 1 + GD))
        def _s4():
            gather_copy(bk).wait()
            scatter_copy(bk)

        return carry

    lax.fori_loop(0, NCH0 + 1 + GD, step, 0)

    for b in range(NB):
        scatter_wait(b)
    plsc.subcore_barrier()

    if out_padded:
        pltpu.sync_copy(acc.at[pl.ds(base, ROWS_PER_TILE)],
                        out_hbm.at[pl.ds(c_p + base, ROWS_PER_TILE)])
    else:
        c_pr = c * PR

        @pl.when(s < NS - 1)
        def _full():
            pltpu.sync_copy(acc.at[pl.ds(base, ROWS_PER_TILE)],
                            out_hbm.at[pl.ds(c_pr + base, ROWS_PER_TILE)])

        @pl.when(s == NS - 1)
        def _last():
            pltpu.sync_copy(acc.at[pl.ds(base, OUT_LAST)],
                            out_hbm.at[pl.ds(c_pr + base, OUT_LAST)])


def _make_sc(remap_src, out_padded):
    out_rows = NP if out_padded else N
    return functools.partial(
        pl.kernel,
        out_type=jax.ShapeDtypeStruct((out_rows, D), jnp.float32),
        mesh=plsc.VectorSubcoreMesh(
            core_axis_name="c", subcore_axis_name="s", num_cores=NC,
            num_subcores=NS),
        compiler_params=pltpu.CompilerParams(use_tc_tiling_on_sc=False),
        scratch_types=[
            pltpu.VMEM((NB, K), jnp.int32),
            pltpu.VMEM((NB, K), jnp.int32),
            pltpu.VMEM((NB, K, D), jnp.float32),
            pltpu.VMEM((BR, D), jnp.float32),
            pltpu.VMEM((D,), jnp.float32),
            pltpu.SemaphoreType.DMA((NB,)),
            pltpu.SemaphoreType.DMA((NB,)),
            pltpu.SemaphoreType.DMA((NB,)),
            pltpu.VMEM_SHARED((P, D), jnp.float32),
        ],
    )(functools.partial(_sc_body, remap_src=remap_src, out_padded=out_padded))


_sc_layer1 = _make_sc(remap_src=False, out_padded=True)
_sc_layer2 = _make_sc(remap_src=True, out_padded=False)


def kernel(x, edge_index, W1, b1, W2, b2):
    src = edge_index[0].astype(jnp.int32)
    dst = edge_index[1].astype(jnp.int32)
    pad = E_PAD - E
    pad_src = jnp.arange(pad, dtype=jnp.int32)
    src = jnp.concatenate([src, pad_src])
    dst = jnp.concatenate([dst, jnp.full((pad,), BIG, jnp.int32)])

    m1 = _matmul(x, W1, act=False, bm=400)
    agg1 = _sc_layer1(src, dst, m1, b1)
    m2 = _matmul(agg1, W2, act=True, bm=512)
    return _sc_layer2(src, dst, m2, b2)


# speedup vs baseline: 9.9082x; 1.1816x over previous
import functools

import jax
import jax.numpy as jnp
from jax import lax
from jax.experimental import pallas as pl
from jax.experimental.pallas import tpu as pltpu
from jax.experimental.pallas import tpu_sc as plsc

N = 50000
D = 64
E = 800000

NC = 2
NS = 16
L = 16

PR = 25000
P = 25088
GAP = P - PR
NP = NC * P

K = 128
EP0 = 50048
NCH0 = EP0 // K

NB = 3
GD = 1

BR = 64
ROWS_PER_TILE = P // NS
OUT_LAST = PR - 15 * ROWS_PER_TILE


def _mm_body(x_ref, w_ref, o_ref, *, act):
    x = x_ref[...]
    if act:
        x = x * jax.nn.sigmoid(x)
    o_ref[...] = jnp.dot(x, w_ref[...], preferred_element_type=jnp.float32)


def _matmul(x, w, act, bm):
    m = x.shape[0]
    return pl.pallas_call(
        functools.partial(_mm_body, act=act),
        grid=(m // bm,),
        in_specs=[
            pl.BlockSpec((bm, D), lambda i: (i, 0)),
            pl.BlockSpec((D, D), lambda i: (0, 0)),
        ],
        out_specs=pl.BlockSpec((bm, D), lambda i: (i, 0)),
        out_shape=jax.ShapeDtypeStruct((m, D), jnp.float32),
    )(x, w)


def _sc_body(ei_hbm, m_hbm, bias_hbm, out_hbm,
             idx_src, idx_dst, rows_v, blk_v, bias_v,
             sem_i, sem_g, sem_s, acc, *, remap_src, out_padded):
    c = lax.axis_index("c")
    s = lax.axis_index("s")
    c_p = c * P

    pltpu.sync_copy(bias_hbm, bias_v)
    bvecs = [bias_v[pl.ds(l * L, L)] for l in range(D // L)]

    def fill_row(r, carry):
        for l in range(D // L):
            blk_v[r, pl.ds(l * L, L)] = bvecs[l]
        return carry

    lax.fori_loop(0, BR, fill_row, 0)

    base = s * ROWS_PER_TILE
    for j in range(ROWS_PER_TILE // BR):
        pltpu.sync_copy(blk_v, acc.at[pl.ds(base + j * BR, BR)])
    rem = ROWS_PER_TILE % BR
    if rem:
        pltpu.sync_copy(blk_v.at[pl.ds(0, rem)],
                        acc.at[pl.ds(base + (ROWS_PER_TILE // BR) * BR, rem)])
    plsc.subcore_barrier()

    ebase = s * EP0
    nch = NCH0
    lane = lax.iota(jnp.int32, L)

    def idx_copies(chunk, b):
        off = jnp.minimum(ebase + chunk * K, E - K)
        return (pltpu.make_async_copy(ei_hbm.at[0, pl.ds(off, K)],
                                      idx_src.at[b], sem_i.at[b]),
                pltpu.make_async_copy(ei_hbm.at[1, pl.ds(off, K)],
                                      idx_dst.at[b], sem_i.at[b]))

    def gather_copy(b):
        return pltpu.make_async_copy(m_hbm.at[idx_src.at[b]],
                                     rows_v.at[b], sem_g.at[b])

    def scatter_copy(b):
        return pltpu.async_copy(rows_v.at[b], acc.at[idx_dst.at[b]],
                                sem_s.at[b], add=True)

    def scatter_wait(b):
        pltpu.make_async_copy(rows_v.at[b], acc.at[idx_dst.at[b]],
                              sem_s.at[b]).wait()

    def step(i, carry):
        b = lax.rem(i, NB)

        @pl.when((i >= NB) & (i < nch))
        def _s1():
            scatter_wait(b)

        @pl.when(i < nch)
        def _s2():
            ca, cb = idx_copies(i, b)
            ca.start()
            cb.start()

        g = i - 1
        bg = lax.rem(g + NB, NB)

        oob = jnp.where(ebase + g * K < E, 0, 1 << 27)

        @pl.when((i >= 1) & (i < nch + 1))
        def _s3():
            ca, cb = idx_copies(g, bg)
            ca.wait()
            cb.wait()
            for v in range(K // L):
                sl = pl.ds(v * L, L)
                if remap_src:
                    sv = idx_src[bg, sl]
                    sv = sv + jnp.where(sv >= PR, GAP, 0)
                    idx_src[bg, sl] = sv
                dv = idx_dst[bg, sl]
                dv = dv + jnp.where(dv >= PR, GAP, 0) - c_p + oob
                valid = (dv >= 0) & (dv < PR)
                dummy = (PR + 24 + v * 8) + (lane & 7)
                dv = jnp.where(valid, dv, dummy)
                idx_dst[bg, sl] = dv
            gather_copy(bg).start()

        k = i - 1 - GD
        bk = lax.rem(k + NB, NB)

        @pl.when((i >= 1 + GD) & (i < nch + 1 + GD))
        def _s4():
            gather_copy(bk).wait()
            scatter_copy(bk)

        return carry

    lax.fori_loop(0, NCH0 + 1 + GD, step, 0)

    for b in range(NB):
        scatter_wait(b)
    plsc.subcore_barrier()

    if out_padded:
        pltpu.sync_copy(acc.at[pl.ds(base, ROWS_PER_TILE)],
                        out_hbm.at[pl.ds(c_p + base, ROWS_PER_TILE)])
    else:
        c_pr = c * PR

        @pl.when(s < NS - 1)
        def _full():
            pltpu.sync_copy(acc.at[pl.ds(base, ROWS_PER_TILE)],
                            out_hbm.at[pl.ds(c_pr + base, ROWS_PER_TILE)])

        @pl.when(s == NS - 1)
        def _last():
            pltpu.sync_copy(acc.at[pl.ds(base, OUT_LAST)],
                            out_hbm.at[pl.ds(c_pr + base, OUT_LAST)])


def _make_sc(remap_src, out_padded):
    out_rows = NP if out_padded else N
    return functools.partial(
        pl.kernel,
        out_type=jax.ShapeDtypeStruct((out_rows, D), jnp.float32),
        mesh=plsc.VectorSubcoreMesh(
            core_axis_name="c", subcore_axis_name="s", num_cores=NC,
            num_subcores=NS),
        compiler_params=pltpu.CompilerParams(use_tc_tiling_on_sc=False),
        scratch_types=[
            pltpu.VMEM((NB, K), jnp.int32),
            pltpu.VMEM((NB, K), jnp.int32),
            pltpu.VMEM((NB, K, D), jnp.float32),
            pltpu.VMEM((BR, D), jnp.float32),
            pltpu.VMEM((D,), jnp.float32),
            pltpu.SemaphoreType.DMA((NB,)),
            pltpu.SemaphoreType.DMA((NB,)),
            pltpu.SemaphoreType.DMA((NB,)),
            pltpu.VMEM_SHARED((P, D), jnp.float32),
        ],
    )(functools.partial(_sc_body, remap_src=remap_src, out_padded=out_padded))


_sc_layer1 = _make_sc(remap_src=False, out_padded=True)
_sc_layer2 = _make_sc(remap_src=True, out_padded=False)


def kernel(x, edge_index, W1, b1, W2, b2):
    ei = edge_index.astype(jnp.int32)

    m1 = _matmul(x, W1, act=False, bm=10000)
    agg1 = _sc_layer1(ei, m1, b1)
    m2 = _matmul(agg1, W2, act=True, bm=7168)
    return _sc_layer2(ei, m2, b2)


# speedup vs baseline: 11.9575x; 1.2068x over previous
import functools

import jax
import jax.numpy as jnp
from jax import lax
from jax.experimental import pallas as pl
from jax.experimental.pallas import tpu as pltpu
from jax.experimental.pallas import tpu_sc as plsc

N = 50000
D = 64
E = 800000

NC = 2
NS = 16
L = 16

PR = 25000
P = 25088
GAP = P - PR
NP = NC * P

K = 128
EP0 = 50048
NCH0 = EP0 // K

NR = 2
NB = 3
SB = 272

BR = 32
ROWS_PER_TILE = P // NS
OUT_LAST = PR - 15 * ROWS_PER_TILE


def _mm_body(x_ref, w_ref, o_ref, *, act):
    x = x_ref[...]
    if act:
        x = x * jax.nn.sigmoid(x)
    o_ref[...] = jnp.dot(x, w_ref[...], preferred_element_type=jnp.float32)


def _matmul(x, w, act, bm):
    m = x.shape[0]
    return pl.pallas_call(
        functools.partial(_mm_body, act=act),
        grid=(m // bm,),
        in_specs=[
            pl.BlockSpec((bm, D), lambda i: (i, 0)),
            pl.BlockSpec((D, D), lambda i: (0, 0)),
        ],
        out_specs=pl.BlockSpec((bm, D), lambda i: (i, 0)),
        out_shape=jax.ShapeDtypeStruct((m, D), jnp.float32),
    )(x, w)


def _sc_body(ei_hbm, m_hbm, bias_hbm, out_hbm,
             raw_src, raw_dst, idx_src, idx_dst, rows_v,
             ssrc, sdst, blk_v, bias_v,
             sem_r, sem_g, sem_s, acc, *, remap_src, out_padded):
    c = lax.axis_index("c")
    s = lax.axis_index("s")
    c_p = c * P

    pltpu.sync_copy(bias_hbm, bias_v)
    bvecs = [bias_v[pl.ds(l * L, L)] for l in range(D // L)]

    def fill_row(r, carry):
        for l in range(D // L):
            blk_v[r, pl.ds(l * L, L)] = bvecs[l]
        return carry

    lax.fori_loop(0, BR, fill_row, 0)

    base = s * ROWS_PER_TILE
    for j in range(ROWS_PER_TILE // BR):
        pltpu.sync_copy(blk_v, acc.at[pl.ds(base + j * BR, BR)])
    plsc.subcore_barrier()

    ebase = s * EP0
    nch = NCH0
    lane = lax.iota(jnp.int32, L)

    def raw_copies(chunk, b):
        off = jnp.minimum(ebase + chunk * K, E - K)
        return (pltpu.make_async_copy(ei_hbm.at[0, pl.ds(off, K)],
                                      raw_src.at[b], sem_r.at[b]),
                pltpu.make_async_copy(ei_hbm.at[1, pl.ds(off, K)],
                                      raw_dst.at[b], sem_r.at[b]))

    def gather_copy(b):
        return pltpu.make_async_copy(m_hbm.at[idx_src.at[b]],
                                     rows_v.at[b], sem_g.at[b])

    def scatter_fire(b):
        pltpu.async_copy(rows_v.at[b], acc.at[idx_dst.at[b]],
                         sem_s.at[b], add=True)

    def scatter_wait(b):
        pltpu.make_async_copy(rows_v.at[b], acc.at[idx_dst.at[b]],
                              sem_s.at[b]).wait()

    def do_fire(f):
        bf = lax.rem(f, NB)

        @pl.when(f >= NB)
        def _free_slot():
            scatter_wait(bf)

        for v in range(K // L):
            sl = pl.ds(v * L, L)
            idx_src[bf, sl] = ssrc[sl]
            idx_dst[bf, sl] = sdst[sl]
        gather_copy(bf).start()

        @pl.when(f >= 1)
        def _chase():
            bp = lax.rem(f - 1 + NB, NB)
            gather_copy(bp).wait()
            scatter_fire(bp)

    def step(i, carry):
        fill, nf = carry

        @pl.when(i < nch)
        def _prefetch():
            ca, cb = raw_copies(i, lax.rem(i, NR))
            ca.start()
            cb.start()

        g = i - 1
        bg = lax.rem(g, NR)
        oob = jnp.where(ebase + g * K < E, 0, 1 << 27)

        ca, cb = raw_copies(g, bg)
        ca.wait()
        cb.wait()
        for v in range(K // L):
            sl = pl.ds(v * L, L)
            sv = raw_src[bg, sl]
            if remap_src:
                sv = sv + jnp.where(sv >= PR, GAP, 0)
            dv = raw_dst[bg, sl]
            dv = dv + jnp.where(dv >= PR, GAP, 0) - c_p + oob
            valid = (dv >= 0) & (dv < PR)
            vi = valid.astype(jnp.int32)
            pc = plsc.cumsum(vi)
            posv = fill + pc - 1
            plsc.store_scatter(ssrc, [posv], sv, mask=valid)
            plsc.store_scatter(sdst, [posv], dv, mask=valid)
            fill = fill + jnp.sum(vi)

        @pl.when(fill >= K)
        def _fire():
            do_fire(nf)
            for v in range(K // L):
                lo = pl.ds(v * L, L)
                hi = pl.ds(K + v * L, L)
                ssrc[lo] = ssrc[hi]
                sdst[lo] = sdst[hi]

        fired = jnp.where(fill >= K, 1, 0)
        return fill - fired * K, nf + fired

    ca0, cb0 = raw_copies(0, 0)
    ca0.start()
    cb0.start()
    fill, nf = lax.fori_loop(1, nch + 1, step,
                             (jnp.int32(0), jnp.int32(0)))

    for v in range(K // L):
        sl = pl.ds(v * L, L)
        pos = v * L + lane
        pad_dst = (PR + 24 + v * 8) + (lane & 7)
        ssrc[sl] = jnp.where(pos < fill, ssrc[sl], 0)
        sdst[sl] = jnp.where(pos < fill, sdst[sl], pad_dst)
    do_fire(nf)
    bl = lax.rem(nf, NB)
    gather_copy(bl).wait()
    scatter_fire(bl)

    for j in range(NB):
        f = nf - j

        @pl.when(f >= 0)
        def _drain(f=f):
            scatter_wait(lax.rem(f + NB * NCH0, NB))

    plsc.subcore_barrier()

    if out_padded:
        pltpu.sync_copy(acc.at[pl.ds(base, ROWS_PER_TILE)],
                        out_hbm.at[pl.ds(c_p + base, ROWS_PER_TILE)])
    else:
        c_pr = c * PR

        @pl.when(s < NS - 1)
        def _full():
            pltpu.sync_copy(acc.at[pl.ds(base, ROWS_PER_TILE)],
                            out_hbm.at[pl.ds(c_pr + base, ROWS_PER_TILE)])

        @pl.when(s == NS - 1)
        def _last():
            pltpu.sync_copy(acc.at[pl.ds(base, OUT_LAST)],
                            out_hbm.at[pl.ds(c_pr + base, OUT_LAST)])


def _make_sc(remap_src, out_padded):
    out_rows = NP if out_padded else N
    return functools.partial(
        pl.kernel,
        out_type=jax.ShapeDtypeStruct((out_rows, D), jnp.float32),
        mesh=plsc.VectorSubcoreMesh(
            core_axis_name="c", subcore_axis_name="s", num_cores=NC,
            num_subcores=NS),
        compiler_params=pltpu.CompilerParams(use_tc_tiling_on_sc=False, needs_layout_passes=False),
        scratch_types=[
            pltpu.VMEM((NR, K), jnp.int32),
            pltpu.VMEM((NR, K), jnp.int32),
            pltpu.VMEM((NB, K), jnp.int32),
            pltpu.VMEM((NB, K), jnp.int32),
            pltpu.VMEM((NB, K, D), jnp.float32),
            pltpu.VMEM((SB,), jnp.int32),
            pltpu.VMEM((SB,), jnp.int32),
            pltpu.VMEM((BR, D), jnp.float32),
            pltpu.VMEM((D,), jnp.float32),
            pltpu.SemaphoreType.DMA((NR,)),
            pltpu.SemaphoreType.DMA((NB,)),
            pltpu.SemaphoreType.DMA((NB,)),
            pltpu.VMEM_SHARED((P, D), jnp.float32),
        ],
    )(functools.partial(_sc_body, remap_src=remap_src, out_padded=out_padded))


_sc_layer1 = _make_sc(remap_src=False, out_padded=True)
_sc_layer2 = _make_sc(remap_src=True, out_padded=False)


def kernel(x, edge_index, W1, b1, W2, b2):
    ei = edge_index.astype(jnp.int32)

    m1 = _matmul(x, W1, act=False, bm=10000)
    agg1 = _sc_layer1(ei, m1, b1)
    m2 = _matmul(agg1, W2, act=True, bm=7168)
    return _sc_layer2(ei, m2, b2)


# speedup vs baseline: 12.2956x; 1.0283x over previous
import functools

import jax
import jax.numpy as jnp
from jax import lax
from jax.experimental import pallas as pl
from jax.experimental.pallas import tpu as pltpu
from jax.experimental.pallas import tpu_sc as plsc

N = 50000
D = 64
E = 800000

NC = 2
NS = 16
L = 16

PR = 25000
P = 25088
GAP = P - PR
NP = NC * P

K = 128
EP0 = 50048
NCH0 = EP0 // K

NR = 2
NB = 3
SB = 272

BR = 32
ROWS_PER_TILE = P // NS
OUT_LAST = PR - 15 * ROWS_PER_TILE


def _mm_body(x_ref, w_ref, o_ref, *, act):
    x = x_ref[...]
    if act:
        x = x * jax.nn.sigmoid(x)
    o_ref[...] = jnp.dot(x, w_ref[...], preferred_element_type=jnp.float32)


def _matmul(x, w, act, bm):
    m = x.shape[0]
    return pl.pallas_call(
        functools.partial(_mm_body, act=act),
        grid=(m // bm,),
        in_specs=[
            pl.BlockSpec((bm, D), lambda i: (i, 0)),
            pl.BlockSpec((D, D), lambda i: (0, 0)),
        ],
        out_specs=pl.BlockSpec((bm, D), lambda i: (i, 0)),
        out_shape=jax.ShapeDtypeStruct((m, D), jnp.float32),
    )(x, w)


def _sc_body(ei_hbm, m_hbm, bias_hbm, out_hbm,
             raw_src, raw_dst, idx_src, idx_dst, rows_v,
             ssrc, sdst, blk_v, bias_v,
             sem_r, sem_g, sem_s, acc, *, remap_src, out_padded):
    c = lax.axis_index("c")
    s = lax.axis_index("s")
    c_p = c * P

    pltpu.sync_copy(bias_hbm, bias_v)
    bvecs = [bias_v[pl.ds(l * L, L)] for l in range(D // L)]

    def fill_row(r, carry):
        for l in range(D // L):
            blk_v[r, pl.ds(l * L, L)] = bvecs[l]
        return carry

    lax.fori_loop(0, BR, fill_row, 0)

    base = s * ROWS_PER_TILE
    for j in range(ROWS_PER_TILE // BR):
        pltpu.sync_copy(blk_v, acc.at[pl.ds(base + j * BR, BR)])
    plsc.subcore_barrier()

    ebase = s * EP0
    nch = NCH0
    lane = lax.iota(jnp.int32, L)

    def raw_copies(chunk, b):
        off = jnp.minimum(ebase + chunk * K, E - K)
        return (pltpu.make_async_copy(ei_hbm.at[0, pl.ds(off, K)],
                                      raw_src.at[b], sem_r.at[b]),
                pltpu.make_async_copy(ei_hbm.at[1, pl.ds(off, K)],
                                      raw_dst.at[b], sem_r.at[b]))

    def gather_copy(b):
        return pltpu.make_async_copy(m_hbm.at[idx_src.at[b]],
                                     rows_v.at[b], sem_g.at[b])

    def scatter_fire(b):
        pltpu.async_copy(rows_v.at[b], acc.at[idx_dst.at[b]],
                         sem_s.at[b], add=True)

    def scatter_wait(b):
        pltpu.make_async_copy(rows_v.at[b], acc.at[idx_dst.at[b]],
                              sem_s.at[b]).wait()

    def do_fire(f):
        bf = lax.rem(f, NB)

        @pl.when(f >= NB)
        def _free_slot():
            scatter_wait(bf)

        for v in range(K // L):
            sl = pl.ds(v * L, L)
            idx_src[bf, sl] = ssrc[sl]
            idx_dst[bf, sl] = sdst[sl]
        gather_copy(bf).start()

        @pl.when(f >= 1)
        def _chase():
            bp = lax.rem(f - 1 + NB, NB)
            gather_copy(bp).wait()
            scatter_fire(bp)

    def step(i, carry):
        fill, nf = carry

        @pl.when(i < nch)
        def _prefetch():
            ca, cb = raw_copies(i, lax.rem(i, NR))
            ca.start()
            cb.start()

        g = i - 1
        bg = lax.rem(g, NR)
        oob = jnp.where(ebase + g * K < E, 0, 1 << 27)

        ca, cb = raw_copies(g, bg)
        ca.wait()
        cb.wait()
        svs, dvs, vis, cnts = [], [], [], []
        for v in range(K // L):
            sl = pl.ds(v * L, L)
            sv = raw_src[bg, sl]
            if remap_src:
                sv = sv + jnp.where(sv >= PR, GAP, 0)
            dv = raw_dst[bg, sl]
            dv = dv + jnp.where(dv >= PR, GAP, 0) - c_p + oob
            valid = (dv >= 0) & (dv < PR)
            vi = valid.astype(jnp.int32)
            svs.append(sv)
            dvs.append(dv)
            vis.append(vi)
            cnts.append(jnp.sum(vi))
        bases = [fill]
        for v in range(K // L):
            bases.append(bases[-1] + cnts[v])
        for v in range(K // L):
            valid = vis[v] > 0
            posv = bases[v] + plsc.cumsum(vis[v]) - 1
            plsc.store_scatter(ssrc, [posv], svs[v], mask=valid)
            plsc.store_scatter(sdst, [posv], dvs[v], mask=valid)
        fill = bases[K // L]

        @pl.when(fill >= K)
        def _fire():
            do_fire(nf)
            for v in range(K // L):
                lo = pl.ds(v * L, L)
                hi = pl.ds(K + v * L, L)
                ssrc[lo] = ssrc[hi]
                sdst[lo] = sdst[hi]

        fired = jnp.where(fill >= K, 1, 0)
        return fill - fired * K, nf + fired

    ca0, cb0 = raw_copies(0, 0)
    ca0.start()
    cb0.start()
    fill, nf = lax.fori_loop(1, nch + 1, step,
                             (jnp.int32(0), jnp.int32(0)))

    for v in range(K // L):
        sl = pl.ds(v * L, L)
        pos = v * L + lane
        pad_dst = (PR + 24 + v * 8) + (lane & 7)
        ssrc[sl] = jnp.where(pos < fill, ssrc[sl], 0)
        sdst[sl] = jnp.where(pos < fill, sdst[sl], pad_dst)
    do_fire(nf)
    bl = lax.rem(nf, NB)
    gather_copy(bl).wait()
    scatter_fire(bl)

    for j in range(NB):
        f = nf - j

        @pl.when(f >= 0)
        def _drain(f=f):
            scatter_wait(lax.rem(f + NB * NCH0, NB))

    plsc.subcore_barrier()

    if out_padded:
        pltpu.sync_copy(acc.at[pl.ds(base, ROWS_PER_TILE)],
                        out_hbm.at[pl.ds(c_p + base, ROWS_PER_TILE)])
    else:
        c_pr = c * PR

        @pl.when(s < NS - 1)
        def _full():
            pltpu.sync_copy(acc.at[pl.ds(base, ROWS_PER_TILE)],
                            out_hbm.at[pl.ds(c_pr + base, ROWS_PER_TILE)])

        @pl.when(s == NS - 1)
        def _last():
            pltpu.sync_copy(acc.at[pl.ds(base, OUT_LAST)],
                            out_hbm.at[pl.ds(c_pr + base, OUT_LAST)])


def _make_sc(remap_src, out_padded):
    out_rows = NP if out_padded else N
    return functools.partial(
        pl.kernel,
        out_type=jax.ShapeDtypeStruct((out_rows, D), jnp.float32),
        mesh=plsc.VectorSubcoreMesh(
            core_axis_name="c", subcore_axis_name="s", num_cores=NC,
            num_subcores=NS),
        compiler_params=pltpu.CompilerParams(use_tc_tiling_on_sc=False, needs_layout_passes=False),
        scratch_types=[
            pltpu.VMEM((NR, K), jnp.int32),
            pltpu.VMEM((NR, K), jnp.int32),
            pltpu.VMEM((NB, K), jnp.int32),
            pltpu.VMEM((NB, K), jnp.int32),
            pltpu.VMEM((NB, K, D), jnp.float32),
            pltpu.VMEM((SB,), jnp.int32),
            pltpu.VMEM((SB,), jnp.int32),
            pltpu.VMEM((BR, D), jnp.float32),
            pltpu.VMEM((D,), jnp.float32),
            pltpu.SemaphoreType.DMA((NR,)),
            pltpu.SemaphoreType.DMA((NB,)),
            pltpu.SemaphoreType.DMA((NB,)),
            pltpu.VMEM_SHARED((P, D), jnp.float32),
        ],
    )(functools.partial(_sc_body, remap_src=remap_src, out_padded=out_padded))


_sc_layer1 = _make_sc(remap_src=False, out_padded=True)
_sc_layer2 = _make_sc(remap_src=True, out_padded=False)


def kernel(x, edge_index, W1, b1, W2, b2):
    ei = edge_index.astype(jnp.int32)

    m1 = _matmul(x, W1, act=False, bm=10000)
    agg1 = _sc_layer1(ei, m1, b1)
    m2 = _matmul(agg1, W2, act=True, bm=7168)
    return _sc_layer2(ei, m2, b2)
